# Initial kernel scaffold; baseline (speedup 1.0000x reference)
import jax
import jax.numpy as jnp
from jax.experimental import pallas as pl


def kernel(x, edge_index, edge_weight, W1, b1, W2, b2, W3, b3):
    raise NotImplementedError("write your pallas kernel here")


# speedup vs baseline: 7.6437x; 7.6437x over previous
import functools

import jax
import jax.numpy as jnp
from jax import lax
from jax.experimental import pallas as pl
from jax.experimental.pallas import tpu as pltpu
from jax.experimental.pallas import tpu_sc as plsc

_TILES = 16
_K = 128
_LANES = 16


def _sc_mesh():
    return plsc.VectorSubcoreMesh(core_axis_name="c", subcore_axis_name="s")


def _sc_degree(dstp, ewp, npad, n_chunks):
    rows_per_tile = npad // _TILES
    zc = 128

    @functools.partial(
        pl.kernel,
        mesh=_sc_mesh(),
        out_type=jax.ShapeDtypeStruct((npad,), jnp.float32),
        scratch_types=[
            pltpu.VMEM((_K,), jnp.int32),
            pltpu.VMEM((_K,), jnp.float32),
            pltpu.VMEM((zc,), jnp.float32),
            pltpu.VMEM_SHARED((npad,), jnp.float32),
        ],
    )
    def degk(dst_hbm, ew_hbm, out_hbm, dst_v, ew_v, zb_v, acc):
        c = lax.axis_index("c")
        s = lax.axis_index("s")

        @pl.when(c == 0)
        def _body():
            for d in range(zc // _LANES):
                zb_v[pl.ds(d * _LANES, _LANES)] = jnp.zeros((_LANES,), jnp.float32)
            for t in range(rows_per_tile // zc):
                pltpu.sync_copy(zb_v, acc.at[pl.ds(s * rows_per_tile + t * zc, zc)])
            plsc.subcore_barrier()

            def chunk_body(ci, carry):
                e0 = (s * n_chunks + ci) * _K
                pltpu.sync_copy(dst_hbm.at[pl.ds(e0, _K)], dst_v)
                pltpu.sync_copy(ew_hbm.at[pl.ds(e0, _K)], ew_v)
                pltpu.sync_copy(ew_v, acc.at[dst_v], add=True)
                return carry

            lax.fori_loop(0, n_chunks, chunk_body, 0)
            plsc.subcore_barrier()
            pltpu.sync_copy(acc.at[pl.ds(s * rows_per_tile, rows_per_tile)],
                            out_hbm.at[pl.ds(s * rows_per_tile, rows_per_tile)])

    return degk(dstp, ewp)


def _sc_aggregate(g2, srcp, dstp, ewp, n, npad, dh, n_chunks):
    rows_per_tile = npad // _TILES
    zc = 128

    @functools.partial(
        pl.kernel,
        mesh=_sc_mesh(),
        out_type=jax.ShapeDtypeStruct((2 * npad, dh), jnp.float32),
        scratch_types=[
            pltpu.VMEM((_K,), jnp.int32),
            pltpu.VMEM((_K,), jnp.int32),
            pltpu.VMEM((_K,), jnp.float32),
            pltpu.VMEM((_K, dh), jnp.float32),
            pltpu.VMEM_SHARED((npad, dh), jnp.float32),
            pltpu.SemaphoreType.DMA,
        ],
    )
    def agg(g_hbm, src_hbm, dst_hbm, ew_hbm, out_hbm,
            src_v, dst_v, ew_v, rows_v, acc, sem):
        c = lax.axis_index("c")
        s = lax.axis_index("s")
        src_off = c * n

        def zero_row(j, carry):
            for d in range(dh // _LANES):
                rows_v[j, pl.ds(d * _LANES, _LANES)] = jnp.zeros((_LANES,), jnp.float32)
            return carry

        lax.fori_loop(0, zc, zero_row, 0)
        for t in range(rows_per_tile // zc):
            pltpu.sync_copy(rows_v.at[pl.ds(0, zc)],
                            acc.at[pl.ds(s * rows_per_tile + t * zc, zc)])
        plsc.subcore_barrier()

        def chunk_body(ci, carry):
            e0 = (s * n_chunks + ci) * _K
            pltpu.sync_copy(src_hbm.at[pl.ds(e0, _K)], src_v)
            pltpu.sync_copy(dst_hbm.at[pl.ds(e0, _K)], dst_v)
            pltpu.sync_copy(ew_hbm.at[pl.ds(e0, _K)], ew_v)
            for d in range(_K // _LANES):
                src_v[pl.ds(d * _LANES, _LANES)] = (
                    src_v[pl.ds(d * _LANES, _LANES)] + src_off)
            pltpu.async_copy(g_hbm.at[src_v], rows_v, sem).wait()

            def scale_block(jj, carry2):
                blk = ew_v[pl.ds(jj * _LANES, _LANES)]
                for l in range(_LANES):
                    w = jnp.full((_LANES,), blk[l], jnp.float32)
                    j = jj * _LANES + l
                    for d in range(dh // _LANES):
                        rows_v[j, pl.ds(d * _LANES, _LANES)] = (
                            rows_v[j, pl.ds(d * _LANES, _LANES)] * w)
                return carry2

            lax.fori_loop(0, _K // _LANES, scale_block, 0)
            pltpu.sync_copy(rows_v, acc.at[dst_v], add=True)
            return carry

        lax.fori_loop(0, n_chunks, chunk_body, 0)
        plsc.subcore_barrier()
        pltpu.sync_copy(acc.at[pl.ds(s * rows_per_tile, rows_per_tile)],
                        out_hbm.at[pl.ds(c * npad + s * rows_per_tile, rows_per_tile)])

    return agg(g2, srcp, dstp, ewp)


def _sc_aggregate_full(g, srcp, dstp, ewp, n, npad, dh, n_chunks2):
    rows_per_tile = npad // _TILES
    zc = 128

    @functools.partial(
        pl.kernel,
        mesh=_sc_mesh(),
        out_type=jax.ShapeDtypeStruct((2 * npad, dh), jnp.float32),
        scratch_types=[
            pltpu.VMEM((_K,), jnp.int32),
            pltpu.VMEM((_K,), jnp.int32),
            pltpu.VMEM((_K,), jnp.float32),
            pltpu.VMEM((_K, dh), jnp.float32),
            pltpu.VMEM_SHARED((npad, dh), jnp.float32),
            pltpu.SemaphoreType.DMA,
        ],
    )
    def agg(g_hbm, src_hbm, dst_hbm, ew_hbm, out_hbm,
            src_v, dst_v, ew_v, rows_v, acc, sem):
        c = lax.axis_index("c")
        s = lax.axis_index("s")

        def zero_row(j, carry):
            for d in range(dh // _LANES):
                rows_v[j, pl.ds(d * _LANES, _LANES)] = jnp.zeros((_LANES,), jnp.float32)
            return carry

        lax.fori_loop(0, zc, zero_row, 0)
        for t in range(rows_per_tile // zc):
            pltpu.sync_copy(rows_v.at[pl.ds(0, zc)],
                            acc.at[pl.ds(s * rows_per_tile + t * zc, zc)])
        plsc.subcore_barrier()

        core_base = c * (_TILES * n_chunks2 * _K)

        def chunk_body(ci, carry):
            e0 = core_base + (s * n_chunks2 + ci) * _K
            pltpu.sync_copy(src_hbm.at[pl.ds(e0, _K)], src_v)
            pltpu.sync_copy(dst_hbm.at[pl.ds(e0, _K)], dst_v)
            pltpu.sync_copy(ew_hbm.at[pl.ds(e0, _K)], ew_v)
            pltpu.async_copy(g_hbm.at[src_v], rows_v, sem).wait()

            def scale_block(jj, carry2):
                blk = ew_v[pl.ds(jj * _LANES, _LANES)]
                for l in range(_LANES):
                    w = jnp.full((_LANES,), blk[l], jnp.float32)
                    j = jj * _LANES + l
                    for d in range(dh // _LANES):
                        rows_v[j, pl.ds(d * _LANES, _LANES)] = (
                            rows_v[j, pl.ds(d * _LANES, _LANES)] * w)
                return carry2

            lax.fori_loop(0, _K // _LANES, scale_block, 0)
            pltpu.sync_copy(rows_v, acc.at[dst_v], add=True)
            return carry

        lax.fori_loop(0, n_chunks2, chunk_body, 0)
        plsc.subcore_barrier()
        pltpu.sync_copy(acc.at[pl.ds(s * rows_per_tile, rows_per_tile)],
                        out_hbm.at[pl.ds(c * npad + s * rows_per_tile, rows_per_tile)])

    return agg(g, srcp, dstp, ewp)


_HI = lax.Precision.HIGHEST


def _tc_first(deg2, x, W1, n, bn):
    d_in = x.shape[1]
    d_hid = W1.shape[1]
    dh = d_hid // 2
    grid = (n // bn,)

    def body(deg_ref, x_ref, w_ref, dis_ref, g_ref):
        dis = lax.rsqrt(deg_ref[...] + 1.0)
        z = jnp.dot(x_ref[...], w_ref[...], precision=_HI)
        g = dis * z
        dis_ref[...] = dis
        g_ref[0] = g[:, :dh]
        g_ref[1] = g[:, dh:]

    return pl.pallas_call(
        body,
        grid=grid,
        in_specs=[
            pl.BlockSpec((bn, 1), lambda i: (i, 0)),
            pl.BlockSpec((bn, d_in), lambda i: (i, 0)),
            pl.BlockSpec((d_in, d_hid), lambda i: (0, 0)),
        ],
        out_specs=[
            pl.BlockSpec((bn, 1), lambda i: (i, 0)),
            pl.BlockSpec((2, bn, dh), lambda i: (0, i, 0)),
        ],
        out_shape=[
            jax.ShapeDtypeStruct((n, 1), jnp.float32),
            jax.ShapeDtypeStruct((2, n, dh), jnp.float32),
        ],
    )(deg2, x, W1)


def _tc_mid(dis, S, g, b2d, W, n, npad, bn, split_out):
    dh_in = g.shape[2]
    d_out = W.shape[1]
    dho = d_out // 2

    def body(dis_ref, s_ref, g_ref, b_ref, w_ref, go_ref):
        dis = dis_ref[...]
        a0 = jnp.maximum(dis * (s_ref[0] + g_ref[0]) + b_ref[0, :dh_in][None, :], 0.0)
        a1 = jnp.maximum(dis * (s_ref[1] + g_ref[1]) + b_ref[0, dh_in:][None, :], 0.0)
        a = jnp.concatenate([a0, a1], axis=1)
        z = jnp.dot(a, w_ref[...], precision=_HI)
        if split_out:
            go_ref[0] = dis * z[:, :dho]
            go_ref[1] = dis * z[:, dho:]
        else:
            go_ref[...] = dis * z

    if split_out:
        out_spec = pl.BlockSpec((2, bn, dho), lambda i: (0, i, 0))
        out_shape = jax.ShapeDtypeStruct((2, n, dho), jnp.float32)
    else:
        out_spec = pl.BlockSpec((bn, d_out), lambda i: (i, 0))
        out_shape = jax.ShapeDtypeStruct((n, d_out), jnp.float32)

    return pl.pallas_call(
        body,
        grid=(n // bn,),
        in_specs=[
            pl.BlockSpec((bn, 1), lambda i: (i, 0)),
            pl.BlockSpec((2, bn, dh_in), lambda i: (0, i, 0)),
            pl.BlockSpec((2, bn, dh_in), lambda i: (0, i, 0)),
            pl.BlockSpec((1, 2 * dh_in), lambda i: (0, 0)),
            pl.BlockSpec((2 * dh_in, d_out), lambda i: (0, 0)),
        ],
        out_specs=out_spec,
        out_shape=out_shape,
    )(dis, S, g, b2d, W)


def _tc_last(dis, S, g, b2d, n, npad, bn):
    d = g.shape[1]

    def body(dis_ref, s_ref, g_ref, b_ref, out_ref):
        dis = dis_ref[...]
        out_ref[...] = dis * (s_ref[0] + s_ref[1] + g_ref[...]) + b_ref[0][None, :]

    return pl.pallas_call(
        body,
        grid=(n // bn,),
        in_specs=[
            pl.BlockSpec((bn, 1), lambda i: (i, 0)),
            pl.BlockSpec((2, bn, d), lambda i: (0, i, 0)),
            pl.BlockSpec((bn, d), lambda i: (i, 0)),
            pl.BlockSpec((1, d), lambda i: (0, 0)),
        ],
        out_specs=pl.BlockSpec((bn, d), lambda i: (i, 0)),
        out_shape=jax.ShapeDtypeStruct((n, d), jnp.float32),
    )(dis, S, g, b2d)


def kernel(x, edge_index, edge_weight, W1, b1, W2, b2, W3, b3):
    n, d_in = x.shape
    d_hid = W1.shape[1]
    d_out = W3.shape[1]
    e = edge_index.shape[1]
    bn = 1000

    ep = -(-e // (2 * _TILES * _K)) * (2 * _TILES * _K)
    n_chunks = ep // (_TILES * _K)
    n_chunks2 = ep // (2 * _TILES * _K)
    pad = ep - e
    npad = -(-n // (_TILES * 128)) * (_TILES * 128)

    src = edge_index[0].astype(jnp.int32)
    dst = edge_index[1].astype(jnp.int32)
    ew = edge_weight.astype(jnp.float32)
    if pad:
        fill = jnp.arange(pad, dtype=jnp.int32) % n
        src = jnp.concatenate([src, fill])
        dst = jnp.concatenate([dst, fill])
        ew = jnp.concatenate([ew, jnp.zeros((pad,), jnp.float32)])

    deg_raw = _sc_degree(dst, ew, npad, n_chunks)
    deg2 = deg_raw.reshape(npad, 1)

    dis, g1 = _tc_first(deg2, x, W1, n, bn)
    dh1 = d_hid // 2
    S1 = _sc_aggregate(g1.reshape(2 * n, dh1), src, dst, ew, n, npad, dh1,
                       n_chunks).reshape(2, npad, dh1)
    g2 = _tc_mid(dis, S1, g1, b1.reshape(1, d_hid), W2, n, npad, bn, True)
    dh2 = d_hid // 2
    S2 = _sc_aggregate(g2.reshape(2 * n, dh2), src, dst, ew, n, npad, dh2,
                       n_chunks).reshape(2, npad, dh2)
    g3 = _tc_mid(dis, S2, g2, b2.reshape(1, d_hid), W3, n, npad, bn, False)
    S3 = _sc_aggregate_full(g3, src, dst, ew, n, npad, d_out,
                            n_chunks2).reshape(2, npad, d_out)
    out = _tc_last(dis, S3, g3, b3.reshape(1, d_out), n, npad, bn)
    return out


# speedup vs baseline: 18.7032x; 2.4469x over previous
import functools

import jax
import jax.numpy as jnp
from jax import lax
from jax.experimental import pallas as pl
from jax.experimental.pallas import tpu as pltpu
from jax.experimental.pallas import tpu_sc as plsc

_TILES = 16
_BK = 128
_HB = _BK // 128
_LANES = 16
_UNROLL = 4


def _sc_mesh():
    return plsc.VectorSubcoreMesh(core_axis_name="c", subcore_axis_name="s")


def _sc_degree(packed, ewp, npad, nc):
    rows_per_tile = npad // _TILES
    zc = 128

    @functools.partial(
        pl.kernel,
        mesh=_sc_mesh(),
        out_type=jax.ShapeDtypeStruct((npad,), jnp.float32),
        scratch_types=(
            [pltpu.VMEM((2 * _HB, 128), jnp.int32) for _ in range(_UNROLL)]
            + [pltpu.VMEM((_BK,), jnp.float32) for _ in range(_UNROLL)]
            + [pltpu.VMEM((zc,), jnp.float32),
               pltpu.VMEM_SHARED((npad,), jnp.float32)]
            + [pltpu.SemaphoreType.DMA for _ in range(_UNROLL + 2)]
        ),
    )
    def degk(pk_hbm, ew_hbm, out_hbm, p0, p1, p2, p3, ew0, ew1, ew2, ew3,
             zb_v, acc, si0, si1, si2, si3, ss0, ss1):
        pb = (p0, p1, p2, p3)
        eb = (ew0, ew1, ew2, ew3)
        si = (si0, si1, si2, si3)
        ss = (ss0, ss1)
        c = lax.axis_index("c")
        s = lax.axis_index("s")

        @pl.when(c == 0)
        def _body():
            base = s * nc

            def idx_cp(ci, p):
                return pltpu.make_async_copy(pk_hbm.at[base + ci], pb[p], si[p])

            def ew_cp(ci, p):
                return pltpu.make_async_copy(
                    ew_hbm.at[pl.ds((base + ci) * _BK, _BK)], eb[p], si[p])

            def scat_cp(p, b, k):
                return pltpu.make_async_copy(
                    eb[p].at[pl.ds(k * 128, 128)],
                    acc.at[pb[p].at[_HB + k]], ss[b])

            for d in range(zc // _LANES):
                zb_v[pl.ds(d * _LANES, _LANES)] = jnp.zeros((_LANES,), jnp.float32)
            for t in range(rows_per_tile // zc):
                pltpu.sync_copy(zb_v, acc.at[pl.ds(s * rows_per_tile + t * zc, zc)])
            plsc.subcore_barrier()

            idx_cp(0, 0).start()
            ew_cp(0, 0).start()
            idx_cp(1, 1).start()
            ew_cp(1, 1).start()

            def super_body(i2, carry):
                for u in range(_UNROLL):
                    ci = i2 * _UNROLL + u
                    b = u % 2
                    pc = u

                    @pl.when(ci + 2 < nc)
                    def _():
                        idx_cp(ci + 2, (u + 2) % _UNROLL).start()
                        ew_cp(ci + 2, (u + 2) % _UNROLL).start()

                    idx_cp(ci, pc).wait()
                    ew_cp(ci, pc).wait()

                    @pl.when(ci > 0)
                    def _():
                        for k in range(_HB):
                            scat_cp((u + 3) % _UNROLL, b ^ 1, k).wait()

                    for k in range(_HB):
                        pltpu.async_copy(
                            eb[pc].at[pl.ds(k * 128, 128)],
                            acc.at[pb[pc].at[_HB + k]], ss[b], add=True)
                return carry

            lax.fori_loop(0, nc // _UNROLL, super_body, 0)
            for k in range(_HB):
                scat_cp((nc - 1) % _UNROLL, (nc - 1) % 2, k).wait()
            plsc.subcore_barrier()
            pltpu.sync_copy(acc.at[pl.ds(s * rows_per_tile, rows_per_tile)],
                            out_hbm.at[pl.ds(s * rows_per_tile, rows_per_tile)])

    return degk(packed, ewp)


def _sc_agg(g2, packed, ewp, n, npad, dh, nc, edge_split):
    rows_per_tile = npad // _TILES
    zc = 128

    @functools.partial(
        pl.kernel,
        mesh=_sc_mesh(),
        out_type=jax.ShapeDtypeStruct((2 * npad, dh), jnp.float32),
        scratch_types=(
            [pltpu.VMEM((2 * _HB, 128), jnp.int32) for _ in range(_UNROLL)]
            + [pltpu.VMEM((_BK, dh), jnp.float32) for _ in range(2)]
            + [pltpu.VMEM((_BK,), jnp.float32) for _ in range(_UNROLL)]
            + [pltpu.VMEM_SHARED((npad, dh), jnp.float32)]
            + [pltpu.SemaphoreType.DMA for _ in range(_UNROLL + 4)]
        ),
    )
    def agg(g_hbm, pk_hbm, ew_hbm, out_hbm, p0, p1, p2, p3, r0, r1,
            ew0, ew1, ew2, ew3, acc, si0, si1, si2, si3, sg0, sg1, ss0, ss1):
        pb = (p0, p1, p2, p3)
        rw = (r0, r1)
        eb = (ew0, ew1, ew2, ew3)
        si = (si0, si1, si2, si3)
        sg = (sg0, sg1)
        ss = (ss0, ss1)
        c = lax.axis_index("c")
        s = lax.axis_index("s")
        if edge_split:
            base = (c * _TILES + s) * nc
        else:
            base = s * nc

        def idx_cp(ci, p):
            return pltpu.make_async_copy(pk_hbm.at[base + ci], pb[p], si[p])

        def ew_cp(ci, p):
            return pltpu.make_async_copy(
                ew_hbm.at[pl.ds((base + ci) * _BK, _BK)], eb[p], si[p])

        def gat_cp(p, b, k):
            return pltpu.make_async_copy(
                g_hbm.at[pb[p].at[k]], rw[b].at[pl.ds(k * 128, 128)], sg[b])

        def scat_cp(p, b, k):
            return pltpu.make_async_copy(
                rw[b].at[pl.ds(k * 128, 128)], acc.at[pb[p].at[_HB + k]],
                ss[b])

        def src_offset(p):
            if not edge_split:
                off = c * n
                for k in range(_HB):
                    for d in range(128 // _LANES):
                        pb[p][k, pl.ds(d * _LANES, _LANES)] = (
                            pb[p][k, pl.ds(d * _LANES, _LANES)] + off)

        def zero_row(j, carry):
            for d in range(dh // _LANES):
                rw[0][j, pl.ds(d * _LANES, _LANES)] = jnp.zeros((_LANES,), jnp.float32)
            return carry

        lax.fori_loop(0, zc, zero_row, 0)
        for t in range(rows_per_tile // zc):
            pltpu.sync_copy(rw[0].at[pl.ds(0, zc)],
                            acc.at[pl.ds(s * rows_per_tile + t * zc, zc)])
        plsc.subcore_barrier()

        idx_cp(0, 0).start()
        ew_cp(0, 0).start()
        idx_cp(1, 1).start()
        ew_cp(1, 1).start()
        idx_cp(2, 2).start()
        ew_cp(2, 2).start()
        idx_cp(0, 0).wait()
        ew_cp(0, 0).wait()
        src_offset(0)
        for k in range(_HB):
            gat_cp(0, 0, k).start()

        def super_body(i2, carry):
            for u in range(_UNROLL):
                ci = i2 * _UNROLL + u
                b = u % 2
                pc = u

                for k in range(_HB):
                    gat_cp(pc, b, k).wait()

                @pl.when(ci > 0)
                def _():
                    for k in range(_HB):
                        scat_cp((u + 3) % _UNROLL, b ^ 1, k).wait()

                @pl.when(ci + 2 < nc)
                def _():
                    idx_cp(ci + 2, (u + 2) % _UNROLL).start()
                    ew_cp(ci + 2, (u + 2) % _UNROLL).start()

                @pl.when(ci + 1 < nc)
                def _():
                    idx_cp(ci + 1, (u + 1) % _UNROLL).wait()
                    ew_cp(ci + 1, (u + 1) % _UNROLL).wait()
                    src_offset((u + 1) % _UNROLL)
                    for k in range(_HB):
                        gat_cp((u + 1) % _UNROLL, b ^ 1, k).start()

                def scale_grp(jj, carry2):
                    blk = eb[pc][pl.ds(jj * _LANES, _LANES)]
                    for l in range(_LANES):
                        w = jnp.full((_LANES,), blk[l], jnp.float32)
                        j = jj * _LANES + l
                        for d in range(dh // _LANES):
                            rw[b][j, pl.ds(d * _LANES, _LANES)] = (
                                rw[b][j, pl.ds(d * _LANES, _LANES)] * w)
                    return carry2

                lax.fori_loop(0, _BK // _LANES, scale_grp, 0)

                for k in range(_HB):
                    pltpu.async_copy(
                        rw[b].at[pl.ds(k * 128, 128)],
                        acc.at[pb[pc].at[_HB + k]], ss[b], add=True)
            return carry

        lax.fori_loop(0, nc // _UNROLL, super_body, 0)
        for k in range(_HB):
            scat_cp((nc - 1) % _UNROLL, (nc - 1) % 2, k).wait()
        plsc.subcore_barrier()
        pltpu.sync_copy(acc.at[pl.ds(s * rows_per_tile, rows_per_tile)],
                        out_hbm.at[pl.ds(c * npad + s * rows_per_tile, rows_per_tile)])

    return agg(g2, packed, ewp)


_HI = lax.Precision.HIGHEST


def _tc_matmul(x, W, n, bn):
    d_in = x.shape[1]
    d_out = W.shape[1]

    def body(x_ref, w_ref, z_ref):
        z_ref[...] = jnp.dot(x_ref[...], w_ref[...], precision=_HI)

    return pl.pallas_call(
        body,
        grid=(n // bn,),
        in_specs=[
            pl.BlockSpec((bn, d_in), lambda i: (i, 0)),
            pl.BlockSpec((d_in, d_out), lambda i: (0, 0)),
        ],
        out_specs=pl.BlockSpec((bn, d_out), lambda i: (i, 0)),
        out_shape=jax.ShapeDtypeStruct((n, d_out), jnp.float32),
    )(x, W)


def _tc_first(deg2, z, n, bn):
    d_hid = z.shape[1]
    dh = d_hid // 2

    def body(deg_ref, z_ref, dis_ref, g_ref):
        dis = lax.rsqrt(deg_ref[...] + 1.0)
        g = dis * z_ref[...]
        dis_ref[...] = dis
        g_ref[0] = g[:, :dh]
        g_ref[1] = g[:, dh:]

    return pl.pallas_call(
        body,
        grid=(n // bn,),
        in_specs=[
            pl.BlockSpec((bn, 1), lambda i: (i, 0)),
            pl.BlockSpec((bn, d_hid), lambda i: (i, 0)),
        ],
        out_specs=[
            pl.BlockSpec((bn, 1), lambda i: (i, 0)),
            pl.BlockSpec((2, bn, dh), lambda i: (0, i, 0)),
        ],
        out_shape=[
            jax.ShapeDtypeStruct((n, 1), jnp.float32),
            jax.ShapeDtypeStruct((2, n, dh), jnp.float32),
        ],
    )(deg2, z)


def _tc_mid(dis, S, g, b2d, W, n, bn, split_out):
    dh_in = g.shape[2]
    d_out = W.shape[1]
    dho = d_out // 2

    def body(dis_ref, s_ref, g_ref, b_ref, w_ref, go_ref):
        dis = dis_ref[...]
        a0 = jnp.maximum(dis * (s_ref[0] + g_ref[0]) + b_ref[0, :dh_in][None, :], 0.0)
        a1 = jnp.maximum(dis * (s_ref[1] + g_ref[1]) + b_ref[0, dh_in:][None, :], 0.0)
        a = jnp.concatenate([a0, a1], axis=1)
        z = jnp.dot(a, w_ref[...], precision=_HI)
        if split_out:
            go_ref[0] = dis * z[:, :dho]
            go_ref[1] = dis * z[:, dho:]
        else:
            go_ref[...] = dis * z

    if split_out:
        out_spec = pl.BlockSpec((2, bn, dho), lambda i: (0, i, 0))
        out_shape = jax.ShapeDtypeStruct((2, n, dho), jnp.float32)
    else:
        out_spec = pl.BlockSpec((bn, d_out), lambda i: (i, 0))
        out_shape = jax.ShapeDtypeStruct((n, d_out), jnp.float32)

    return pl.pallas_call(
        body,
        grid=(n // bn,),
        in_specs=[
            pl.BlockSpec((bn, 1), lambda i: (i, 0)),
            pl.BlockSpec((2, bn, dh_in), lambda i: (0, i, 0)),
            pl.BlockSpec((2, bn, dh_in), lambda i: (0, i, 0)),
            pl.BlockSpec((1, 2 * dh_in), lambda i: (0, 0)),
            pl.BlockSpec((2 * dh_in, d_out), lambda i: (0, 0)),
        ],
        out_specs=out_spec,
        out_shape=out_shape,
    )(dis, S, g, b2d, W)


def _tc_last(dis, S, g, b2d, n, bn):
    d = g.shape[1]

    def body(dis_ref, s_ref, g_ref, b_ref, out_ref):
        dis = dis_ref[...]
        out_ref[...] = dis * (s_ref[0] + s_ref[1] + g_ref[...]) + b_ref[0][None, :]

    return pl.pallas_call(
        body,
        grid=(n // bn,),
        in_specs=[
            pl.BlockSpec((bn, 1), lambda i: (i, 0)),
            pl.BlockSpec((2, bn, d), lambda i: (0, i, 0)),
            pl.BlockSpec((bn, d), lambda i: (i, 0)),
            pl.BlockSpec((1, d), lambda i: (0, 0)),
        ],
        out_specs=pl.BlockSpec((bn, d), lambda i: (i, 0)),
        out_shape=jax.ShapeDtypeStruct((n, d), jnp.float32),
    )(dis, S, g, b2d)


def kernel(x, edge_index, edge_weight, W1, b1, W2, b2, W3, b3):
    n, d_in = x.shape
    d_hid = W1.shape[1]
    d_out = W3.shape[1]
    e = edge_index.shape[1]
    bn = 1000

    grain = 2 * _TILES * _BK * _UNROLL
    ep = -(-e // grain) * grain
    nc = ep // (_TILES * _BK)
    nc2 = ep // (2 * _TILES * _BK)
    pad = ep - e
    npad = -(-n // (_TILES * 128)) * (_TILES * 128)

    src = edge_index[0].astype(jnp.int32)
    dst = edge_index[1].astype(jnp.int32)
    ew = edge_weight.astype(jnp.float32)
    if pad:
        fill = jnp.arange(pad, dtype=jnp.int32) % n
        src = jnp.concatenate([src, fill])
        dst = jnp.concatenate([dst, fill])
        ew = jnp.concatenate([ew, jnp.zeros((pad,), jnp.float32)])

    packed = jnp.concatenate([
        src.reshape(-1, _HB, 128),
        dst.reshape(-1, _HB, 128),
    ], axis=1)

    deg_raw = _sc_degree(packed, ew, npad, nc)
    z1 = _tc_matmul(x, W1, n, bn)
    dis, g1 = _tc_first(deg_raw.reshape(npad, 1), z1, n, bn)
    dh1 = d_hid // 2
    S1 = _sc_agg(g1.reshape(2 * n, dh1), packed, ew, n, npad, dh1, nc,
                 False).reshape(2, npad, dh1)
    g2 = _tc_mid(dis, S1, g1, b1.reshape(1, d_hid), W2, n, bn, True)
    dh2 = d_hid // 2
    S2 = _sc_agg(g2.reshape(2 * n, dh2), packed, ew, n, npad, dh2, nc,
                 False).reshape(2, npad, dh2)
    g3 = _tc_mid(dis, S2, g2, b2.reshape(1, d_hid), W3, n, bn, False)
    S3 = _sc_agg(g3, packed, ew, n, npad, d_out, nc2, True).reshape(2, npad, d_out)
    out = _tc_last(dis, S3, g3, b3.reshape(1, d_out), n, bn)
    return out
